# baseline (speedup 1.0000x reference)
import jax
import jax.numpy as jnp
from jax.experimental import pallas as pl
from jax.experimental.pallas import tpu as pltpu

_T = 64
_BT = 16


def _loss_step(lab_ref, x_ref, out_ref):
    x = x_ref[...]
    sa = jnp.sum(x[..., :_T], axis=-1)
    sb = jnp.sum(x[..., _T:], axis=-1)
    da = sa - 1.0
    db = sb - 1.0
    part = jnp.sum(da * da) + jnp.sum(db * db)
    lab = lab_ref[0, 0, :]
    j = jax.lax.broadcasted_iota(jnp.int32, sa.shape, 1)
    ca = jnp.where(2 * j == lab[:, None], 99.0 - 18.0 * sa, 0.0)
    cb = jnp.where(2 * j + 1 == lab[:, None], 99.0 - 18.0 * sb, 0.0)
    acc = (part + jnp.sum(ca) + jnp.sum(cb)) * (0.5 / _T)
    out_ref[...] = acc.reshape(1, 1, 1)


def kernel(outputs, labels):
    B, C, H, W, T = outputs.shape
    x = outputs.reshape(B, C // 2, 2 * T)
    n_steps = B // _BT
    lab3 = labels.reshape(n_steps, 1, _BT)
    out = pl.pallas_call(
        _loss_step,
        grid=(n_steps,),
        in_specs=[
            pl.BlockSpec((1, 1, _BT), lambda i: (i, 0, 0)),
            pl.BlockSpec((_BT, C // 2, 2 * T), lambda i: (i, 0, 0)),
        ],
        out_specs=pl.BlockSpec((1, 1, 1), lambda i: (i, 0, 0)),
        out_shape=jax.ShapeDtypeStruct((n_steps, 1, 1), jnp.float32),
        compiler_params=pltpu.CompilerParams(
            dimension_semantics=("parallel",)),
    )(lab3, x)
    return jnp.sum(out)
